# Initial kernel scaffold; baseline (speedup 1.0000x reference)
import jax
import jax.numpy as jnp
from jax.experimental import pallas as pl


def kernel(features_s, labels_s, features_t, edge_index_s, edge_index_t, W1, b1, W2, b2, fcW, fcb, dW, db):
    raise NotImplementedError("write your pallas kernel here")


# speedup vs baseline: 6.3062x; 6.3062x over previous
import functools
import jax
import jax.numpy as jnp
from jax import lax
from jax.experimental import pallas as pl
from jax.experimental.pallas import tpu as pltpu
from jax.experimental.pallas import tpu_sc as plsc

N = 10000
E = 320000
D = 128
H = 128
C = 16
NP = 10240
NC = 2
NS = 16
EPT = E // (NC * NS)
CHUNK = 128
NFULL = EPT // CHUNK
TAIL = EPT - NFULL * CHUNK
WBR = 1000

_mesh = plsc.VectorSubcoreMesh(
    core_axis_name="c", subcore_axis_name="s", num_cores=NC, num_subcores=NS)


def _wid(cid, sid):
    return sid * NC + cid


def _bincount_body(src_s, dst_s, src_t, dst_t, ones_hbm, zrows_hbm, out,
                   deg_sh, idx_v, idx_tail_v, ones_v, sem):
    cid = lax.axis_index("c")
    sid = lax.axis_index("s")
    wid = _wid(cid, sid)

    pltpu.sync_copy(ones_hbm, ones_v)
    for a, arr in enumerate((src_s, dst_s, src_t, dst_t)):
        @pl.when(sid < 10)
        def _z(a=a):
            pltpu.sync_copy(zrows_hbm, deg_sh.at[pl.ds(sid * WBR, WBR)])
        plsc.subcore_barrier()

        base = wid * EPT

        def chunk_body(c, _, arr=arr, base=base):
            off = base + c * CHUNK
            pltpu.sync_copy(arr.at[pl.ds(off, CHUNK)], idx_v)
            pltpu.sync_copy(ones_v, deg_sh.at[idx_v], add=True)
            return _

        lax.fori_loop(0, NFULL, chunk_body, 0, unroll=False)
        off = base + NFULL * CHUNK
        pltpu.sync_copy(arr.at[pl.ds(off, TAIL)], idx_tail_v)
        pltpu.sync_copy(ones_v.at[pl.ds(0, TAIL)],
                        deg_sh.at[idx_tail_v], add=True)

        plsc.subcore_barrier()

        @pl.when(sid < 10)
        def _wb(a=a):
            r0 = sid * WBR
            pltpu.sync_copy(deg_sh.at[pl.ds(r0, WBR)],
                            out.at[cid, a, pl.ds(r0, WBR)])
        plsc.subcore_barrier()


_bincount_call = pl.kernel(
    _bincount_body,
    out_type=jax.ShapeDtypeStruct((NC, 4, N, D), jnp.float32),
    mesh=_mesh,
    scratch_types=[
        pltpu.VMEM_SHARED((N, D), jnp.float32),
        pltpu.VMEM((CHUNK,), jnp.int32),
        pltpu.VMEM((TAIL,), jnp.int32),
        pltpu.VMEM((CHUNK, D), jnp.float32),
        pltpu.SemaphoreType.DMA,
    ],
)


def _agg_body(hp_s, hp_t, src_s, dst_s, src_t, dst_t, zrows_hbm, out,
              agg_sh, src_v, dst_v, rows_v, src_tail_v, dst_tail_v,
              rows_tail_v, sem):
    cid = lax.axis_index("c")
    sid = lax.axis_index("s")
    wid = _wid(cid, sid)

    for g, (hp, sarr, darr) in ((0, (hp_s, src_s, dst_s)),
                                (1, (hp_t, src_t, dst_t))):
        @pl.when(sid < 10)
        def _z(g=g):
            pltpu.sync_copy(zrows_hbm,
                            agg_sh.at[pl.ds(sid * WBR, WBR)])
        plsc.subcore_barrier()

        base = wid * EPT

        def chunk_body(c, _, hp=hp, sarr=sarr, darr=darr, base=base):
            off = base + c * CHUNK
            pltpu.sync_copy(sarr.at[pl.ds(off, CHUNK)], src_v)
            pltpu.sync_copy(darr.at[pl.ds(off, CHUNK)], dst_v)
            pltpu.async_copy(hp.at[src_v], rows_v, sem).wait()
            pltpu.sync_copy(rows_v, agg_sh.at[dst_v], add=True)
            return _

        lax.fori_loop(0, NFULL, chunk_body, 0, unroll=False)
        off = base + NFULL * CHUNK
        pltpu.sync_copy(sarr.at[pl.ds(off, TAIL)], src_tail_v)
        pltpu.sync_copy(darr.at[pl.ds(off, TAIL)], dst_tail_v)
        pltpu.async_copy(hp.at[src_tail_v], rows_tail_v, sem).wait()
        pltpu.sync_copy(rows_tail_v, agg_sh.at[dst_tail_v], add=True)

        plsc.subcore_barrier()

        @pl.when(sid < 10)
        def _wb(g=g):
            r0 = sid * WBR
            pltpu.sync_copy(agg_sh.at[pl.ds(r0, WBR)],
                            out.at[cid, pl.ds(g * N + r0, WBR)])
        plsc.subcore_barrier()


_agg_call = pl.kernel(
    _agg_body,
    out_type=jax.ShapeDtypeStruct((NC, 2 * N, D), jnp.float32),
    mesh=_mesh,
    scratch_types=[
        pltpu.VMEM_SHARED((N, D), jnp.float32),
        pltpu.VMEM((CHUNK,), jnp.int32),
        pltpu.VMEM((CHUNK,), jnp.int32),
        pltpu.VMEM((CHUNK, D), jnp.float32),
        pltpu.VMEM((TAIL,), jnp.int32),
        pltpu.VMEM((TAIL,), jnp.int32),
        pltpu.VMEM((TAIL, D), jnp.float32),
        pltpu.SemaphoreType.DMA,
    ],
)


def _norm_kernel(deg_ref, out_ref):
    deg = deg_ref[0, 0] + deg_ref[1, 0]
    degv = jnp.max(deg, axis=-1, keepdims=True)
    out_ref[...] = lax.rsqrt(jnp.maximum(degv, 1.0)).reshape(1, N, 1)


def _norms(deg_parts):
    return pl.pallas_call(
        _norm_kernel,
        grid=(4,),
        in_specs=[pl.BlockSpec((NC, 1, N, D), lambda a: (0, a, 0, 0))],
        out_specs=pl.BlockSpec((1, N, 1), lambda a: (a, 0, 0)),
        out_shape=jax.ShapeDtypeStruct((4, N, 1), jnp.float32),
    )(deg_parts)


RBLK = 2000


def _hp_kernel(x_ref, ns_ref, w_ref, hp_ref):
    x = x_ref[...]
    ns = ns_ref[...]
    hp_ref[...] = jnp.dot(x * ns, w_ref[...],
                          preferred_element_type=jnp.float32)


def _hp(x2, ns_col, w):
    grid = (2 * N) // RBLK
    return pl.pallas_call(
        _hp_kernel,
        grid=(grid,),
        in_specs=[
            pl.BlockSpec((RBLK, D), lambda i: (i, 0)),
            pl.BlockSpec((RBLK, 1), lambda i: (i, 0)),
            pl.BlockSpec((D, H), lambda i: (0, 0)),
        ],
        out_specs=pl.BlockSpec((RBLK, H), lambda i: (i, 0)),
        out_shape=jax.ShapeDtypeStruct((2 * N, H), jnp.float32),
    )(x2, ns_col, w)


def _mid_kernel(parts_ref, nd_ref, ns_ref, b_ref, w_ref, h_ref, hp_ref):
    agg = parts_ref[0] + parts_ref[1]
    h = jnp.maximum(agg * nd_ref[...] + b_ref[...], 0.0)
    h_ref[...] = h
    hp_ref[...] = jnp.dot(h * ns_ref[...], w_ref[...],
                          preferred_element_type=jnp.float32)


def _mid(parts, nd_col, ns_col, b, w):
    grid = (2 * N) // RBLK
    return pl.pallas_call(
        _mid_kernel,
        grid=(grid,),
        in_specs=[
            pl.BlockSpec((NC, RBLK, H), lambda i: (0, i, 0)),
            pl.BlockSpec((RBLK, 1), lambda i: (i, 0)),
            pl.BlockSpec((RBLK, 1), lambda i: (i, 0)),
            pl.BlockSpec((1, H), lambda i: (0, 0)),
            pl.BlockSpec((H, H), lambda i: (0, 0)),
        ],
        out_specs=(pl.BlockSpec((RBLK, H), lambda i: (i, 0)),
                   pl.BlockSpec((RBLK, H), lambda i: (i, 0))),
        out_shape=(jax.ShapeDtypeStruct((2 * N, H), jnp.float32),
                   jax.ShapeDtypeStruct((2 * N, H), jnp.float32)),
    )(parts, nd_col, ns_col, b, w)


def _final_kernel(parts_ref, nd_ref, b2_ref, fcw_ref, fcb_ref, dw_ref,
                  db_ref, h1_ref, lab_ref, cls_ref, dom_ref, out_ref):
    i = pl.program_id(0)
    nsteps = pl.num_programs(0)

    agg = parts_ref[0] + parts_ref[1]
    h2 = jnp.maximum(agg * nd_ref[...] + b2_ref[...], 0.0)
    logits = jnp.dot(h2, fcw_ref[...],
                     preferred_element_type=jnp.float32) + fcb_ref[...]
    m = jnp.max(logits, axis=-1, keepdims=True)
    lse = jnp.log(jnp.sum(jnp.exp(logits - m), axis=-1, keepdims=True)) + m
    logp = logits - lse
    onehot = (lax.broadcasted_iota(jnp.int32, (RBLK, C), 1)
              == lab_ref[...]).astype(jnp.float32)
    picked = jnp.sum(logp * onehot, axis=-1, keepdims=True)
    rows = (i * RBLK
            + lax.broadcasted_iota(jnp.int32, (RBLK, 1), 0))
    is_src = rows < N
    cls_part = jnp.sum(jnp.where(is_src, picked, 0.0))

    h1 = h1_ref[...]
    dom = (jnp.dot(h1, dw_ref[0:H], preferred_element_type=jnp.float32)
           + jnp.dot(h2, dw_ref[H:2 * H], preferred_element_type=jnp.float32)
           + jnp.dot(logits, dw_ref[2 * H:2 * H + C],
                     preferred_element_type=jnp.float32)
           + db_ref[...])
    md = jnp.max(dom, axis=-1, keepdims=True)
    lsed = jnp.log(jnp.sum(jnp.exp(dom - md), axis=-1, keepdims=True)) + md
    dlogp = dom - lsed
    picked_d = jnp.where(is_src, dlogp[:, 0:1], dlogp[:, 1:2])
    dom_part = jnp.sum(picked_d)

    @pl.when(i == 0)
    def _init():
        cls_ref[...] = jnp.zeros((1, 1), jnp.float32)
        dom_ref[...] = jnp.zeros((1, 1), jnp.float32)

    cls_ref[...] += jnp.reshape(cls_part, (1, 1))
    dom_ref[...] += jnp.reshape(dom_part, (1, 1))

    @pl.when(i == nsteps - 1)
    def _fin():
        class_loss = -cls_ref[...] / N
        domain_loss = -dom_ref[...] / (2 * N)
        out_ref[...] = class_loss + domain_loss * 0.01


def _final(parts, nd_col, b2, fcW, fcb, dW, db, h1, labels2):
    grid = (2 * N) // RBLK
    outs = pl.pallas_call(
        _final_kernel,
        grid=(grid,),
        in_specs=[
            pl.BlockSpec((NC, RBLK, H), lambda i: (0, i, 0)),
            pl.BlockSpec((RBLK, 1), lambda i: (i, 0)),
            pl.BlockSpec((1, H), lambda i: (0, 0)),
            pl.BlockSpec((H, C), lambda i: (0, 0)),
            pl.BlockSpec((1, C), lambda i: (0, 0)),
            pl.BlockSpec((2 * H + C, 2), lambda i: (0, 0)),
            pl.BlockSpec((1, 2), lambda i: (0, 0)),
            pl.BlockSpec((RBLK, H), lambda i: (i, 0)),
            pl.BlockSpec((RBLK, 1), lambda i: (i, 0)),
        ],
        out_specs=(pl.BlockSpec((1, 1), lambda i: (0, 0)),
                   pl.BlockSpec((1, 1), lambda i: (0, 0)),
                   pl.BlockSpec((1, 1), lambda i: (0, 0))),
        out_shape=(jax.ShapeDtypeStruct((1, 1), jnp.float32),
                   jax.ShapeDtypeStruct((1, 1), jnp.float32),
                   jax.ShapeDtypeStruct((1, 1), jnp.float32)),
    )(parts, nd_col, b2, fcW, fcb, dW, db, h1, labels2)
    return outs[2][0, 0]


def kernel(features_s, labels_s, features_t, edge_index_s, edge_index_t,
           W1, b1, W2, b2, fcW, fcb, dW, db):
    src_s = edge_index_s[0].astype(jnp.int32)
    dst_s = edge_index_s[1].astype(jnp.int32)
    src_t = edge_index_t[0].astype(jnp.int32)
    dst_t = edge_index_t[1].astype(jnp.int32)

    zrows = jnp.zeros((WBR, D), jnp.float32)
    onesr = jnp.ones((CHUNK, D), jnp.float32)

    deg_parts = _bincount_call(src_s, dst_s, src_t, dst_t, onesr, zrows)
    norm4 = _norms(deg_parts)
    ns_col = jnp.concatenate([norm4[0], norm4[2]], axis=0)
    nd_col = jnp.concatenate([norm4[1], norm4[3]], axis=0)

    x2 = jnp.concatenate([features_s, features_t], axis=0)
    hp1 = _hp(x2, ns_col, W1)
    parts1 = _agg_call(hp1[:N], hp1[N:], src_s, dst_s, src_t, dst_t,
                       zrows)
    h1, hp2 = _mid(parts1, nd_col, ns_col, b1.reshape(1, H), W2)
    parts2 = _agg_call(hp2[:N], hp2[N:], src_s, dst_s, src_t, dst_t,
                       zrows)

    labels2 = jnp.concatenate(
        [labels_s.astype(jnp.int32), jnp.zeros((N,), jnp.int32)]
    ).reshape(2 * N, 1)
    return _final(parts2, nd_col, b2.reshape(1, H), fcW, fcb.reshape(1, C),
                  dW, db.reshape(1, 2), h1, labels2)


# speedup vs baseline: 8.4743x; 1.3438x over previous
import functools
import jax
import jax.numpy as jnp
from jax import lax
from jax.experimental import pallas as pl
from jax.experimental.pallas import tpu as pltpu
from jax.experimental.pallas import tpu_sc as plsc

N = 10000
E = 320000
D = 128
H = 128
C = 16
NP = 10240
NC = 2
NS = 16
EPT = E // (NC * NS)
CHUNK = 128
NFULL = EPT // CHUNK
TAIL = EPT - NFULL * CHUNK
WBR = 1000

_mesh = plsc.VectorSubcoreMesh(
    core_axis_name="c", subcore_axis_name="s", num_cores=NC, num_subcores=NS)


def _wid(cid, sid):
    return sid * NC + cid


def _bincount_body(src_s, dst_s, src_t, dst_t, ones_hbm, zrows_hbm, out,
                   deg_sh, idx_v, idx_tail_v, ones_v, sem):
    cid = lax.axis_index("c")
    sid = lax.axis_index("s")
    wid = _wid(cid, sid)

    pltpu.sync_copy(ones_hbm, ones_v)
    for a, arr in enumerate((src_s, dst_s, src_t, dst_t)):
        @pl.when(sid < 10)
        def _z(a=a):
            pltpu.sync_copy(zrows_hbm, deg_sh.at[pl.ds(sid * WBR, WBR)])
        plsc.subcore_barrier()

        base = wid * EPT

        def chunk_body(c, _, arr=arr, base=base):
            off = base + c * CHUNK
            pltpu.sync_copy(arr.at[pl.ds(off, CHUNK)], idx_v)
            pltpu.sync_copy(ones_v, deg_sh.at[idx_v], add=True)
            return _

        lax.fori_loop(0, NFULL, chunk_body, 0, unroll=False)
        off = base + NFULL * CHUNK
        pltpu.sync_copy(arr.at[pl.ds(off, TAIL)], idx_tail_v)
        pltpu.sync_copy(ones_v.at[pl.ds(0, TAIL)],
                        deg_sh.at[idx_tail_v], add=True)

        plsc.subcore_barrier()

        @pl.when(sid < 10)
        def _wb(a=a):
            r0 = sid * WBR
            pltpu.sync_copy(deg_sh.at[pl.ds(r0, WBR)],
                            out.at[cid, a, pl.ds(r0, WBR)])
        plsc.subcore_barrier()


_bincount_call = pl.kernel(
    _bincount_body,
    out_type=jax.ShapeDtypeStruct((NC, 4, N, D), jnp.float32),
    mesh=_mesh,
    scratch_types=[
        pltpu.VMEM_SHARED((N, D), jnp.float32),
        pltpu.VMEM((CHUNK,), jnp.int32),
        pltpu.VMEM((TAIL,), jnp.int32),
        pltpu.VMEM((CHUNK, D), jnp.float32),
        pltpu.SemaphoreType.DMA,
    ],
)


KC = 80
NCH = EPT // KC
UNROLL = 4


def _agg_body(hp_s, hp_t, src_s, dst_s, src_t, dst_t, zrows_hbm, out,
              agg_sh, srcb, dstb, rowsb, semi0, semi1,
              semg0, semg1, semg2, semg3, sems):
    cid = lax.axis_index("c")
    sid = lax.axis_index("s")
    wid = _wid(cid, sid)
    semi = (semi0, semi1)
    semg = (semg0, semg1, semg2, semg3)
    NI = NCH // UNROLL

    for g, (hp, sarr, darr) in ((0, (hp_s, src_s, dst_s)),
                                (1, (hp_t, src_t, dst_t))):
        @pl.when(sid < 10)
        def _z(g=g):
            pltpu.sync_copy(zrows_hbm,
                            agg_sh.at[pl.ds(sid * WBR, WBR)])
        plsc.subcore_barrier()

        base = wid * EPT

        def fire_idx(c, b, p, sarr=sarr, darr=darr, base=base):
            off = base + c * KC
            pltpu.async_copy(sarr.at[pl.ds(off, KC)],
                             srcb.at[p, b], semi[p])
            pltpu.async_copy(darr.at[pl.ds(off, KC)],
                             dstb.at[p, b], semi[p])

        def drain_idx(b, p, sarr=sarr):
            pltpu.make_async_copy(sarr.at[pl.ds(0, KC)],
                                  srcb.at[p, b], semi[p]).wait()
            pltpu.make_async_copy(sarr.at[pl.ds(0, KC)],
                                  dstb.at[p, b], semi[p]).wait()

        def body(i, p, pn, last, hp=hp):
            c0 = i * UNROLL
            if not last:
                for u in range(UNROLL):
                    fire_idx(c0 + UNROLL + u, u, pn)
            else:
                fire_idx(c0 + UNROLL, 0, pn)
            for u in range(UNROLL):
                drain_idx(u, p)
            gds = [pltpu.async_copy(hp.at[srcb.at[p, u]], rowsb.at[u],
                                    semg[u]) for u in range(UNROLL)]
            sds = []
            for u in range(UNROLL):
                gds[u].wait()
                sds.append(pltpu.async_copy(
                    rowsb.at[u], agg_sh.at[dstb.at[p, u]], sems, add=True))
            for sd in sds:
                sd.wait()

        for u in range(UNROLL):
            fire_idx(u, u, 0)

        def pair_body(j, _):
            body(2 * j, 0, 1, False)
            body(2 * j + 1, 1, 0, False)
            return _

        lax.fori_loop(0, (NI - 1) // 2, pair_body, 0, unroll=False)
        body(NI - 1, 0, 1, True)
        drain_idx(0, 1)
        pltpu.async_copy(hp.at[srcb.at[1, 0]], rowsb.at[0], semg[0]).wait()
        pltpu.async_copy(rowsb.at[0], agg_sh.at[dstb.at[1, 0]], sems,
                         add=True).wait()

        plsc.subcore_barrier()

        @pl.when(sid < 10)
        def _wb(g=g):
            r0 = sid * WBR
            pltpu.sync_copy(agg_sh.at[pl.ds(r0, WBR)],
                            out.at[cid, pl.ds(g * N + r0, WBR)])
        plsc.subcore_barrier()


_agg_call = pl.kernel(
    _agg_body,
    out_type=jax.ShapeDtypeStruct((NC, 2 * N, D), jnp.float32),
    mesh=_mesh,
    scratch_types=[
        pltpu.VMEM_SHARED((N, D), jnp.float32),
        pltpu.VMEM((2, UNROLL, KC), jnp.int32),
        pltpu.VMEM((2, UNROLL, KC), jnp.int32),
        pltpu.VMEM((UNROLL, KC, D), jnp.float32),
        pltpu.SemaphoreType.DMA,
        pltpu.SemaphoreType.DMA,
        pltpu.SemaphoreType.DMA,
        pltpu.SemaphoreType.DMA,
        pltpu.SemaphoreType.DMA,
        pltpu.SemaphoreType.DMA,
        pltpu.SemaphoreType.DMA,
    ],
)


def _norm_kernel(deg_ref, out_ref):
    deg = deg_ref[0, 0] + deg_ref[1, 0]
    degv = jnp.max(deg, axis=-1, keepdims=True)
    out_ref[...] = lax.rsqrt(jnp.maximum(degv, 1.0)).reshape(1, N, 1)


def _norms(deg_parts):
    return pl.pallas_call(
        _norm_kernel,
        grid=(4,),
        in_specs=[pl.BlockSpec((NC, 1, N, D), lambda a: (0, a, 0, 0))],
        out_specs=pl.BlockSpec((1, N, 1), lambda a: (a, 0, 0)),
        out_shape=jax.ShapeDtypeStruct((4, N, 1), jnp.float32),
    )(deg_parts)


RBLK = 2000


def _hp_kernel(x_ref, ns_ref, w_ref, hp_ref):
    x = x_ref[...]
    ns = ns_ref[...]
    hp_ref[...] = jnp.dot(x * ns, w_ref[...],
                          preferred_element_type=jnp.float32)


def _hp(x2, ns_col, w):
    grid = (2 * N) // RBLK
    return pl.pallas_call(
        _hp_kernel,
        grid=(grid,),
        in_specs=[
            pl.BlockSpec((RBLK, D), lambda i: (i, 0)),
            pl.BlockSpec((RBLK, 1), lambda i: (i, 0)),
            pl.BlockSpec((D, H), lambda i: (0, 0)),
        ],
        out_specs=pl.BlockSpec((RBLK, H), lambda i: (i, 0)),
        out_shape=jax.ShapeDtypeStruct((2 * N, H), jnp.float32),
    )(x2, ns_col, w)


def _mid_kernel(parts_ref, nd_ref, ns_ref, b_ref, w_ref, h_ref, hp_ref):
    agg = parts_ref[0] + parts_ref[1]
    h = jnp.maximum(agg * nd_ref[...] + b_ref[...], 0.0)
    h_ref[...] = h
    hp_ref[...] = jnp.dot(h * ns_ref[...], w_ref[...],
                          preferred_element_type=jnp.float32)


def _mid(parts, nd_col, ns_col, b, w):
    grid = (2 * N) // RBLK
    return pl.pallas_call(
        _mid_kernel,
        grid=(grid,),
        in_specs=[
            pl.BlockSpec((NC, RBLK, H), lambda i: (0, i, 0)),
            pl.BlockSpec((RBLK, 1), lambda i: (i, 0)),
            pl.BlockSpec((RBLK, 1), lambda i: (i, 0)),
            pl.BlockSpec((1, H), lambda i: (0, 0)),
            pl.BlockSpec((H, H), lambda i: (0, 0)),
        ],
        out_specs=(pl.BlockSpec((RBLK, H), lambda i: (i, 0)),
                   pl.BlockSpec((RBLK, H), lambda i: (i, 0))),
        out_shape=(jax.ShapeDtypeStruct((2 * N, H), jnp.float32),
                   jax.ShapeDtypeStruct((2 * N, H), jnp.float32)),
    )(parts, nd_col, ns_col, b, w)


def _final_kernel(parts_ref, nd_ref, b2_ref, fcw_ref, fcb_ref, dw_ref,
                  db_ref, h1_ref, lab_ref, cls_ref, dom_ref, out_ref):
    i = pl.program_id(0)
    nsteps = pl.num_programs(0)

    agg = parts_ref[0] + parts_ref[1]
    h2 = jnp.maximum(agg * nd_ref[...] + b2_ref[...], 0.0)
    logits = jnp.dot(h2, fcw_ref[...],
                     preferred_element_type=jnp.float32) + fcb_ref[...]
    m = jnp.max(logits, axis=-1, keepdims=True)
    lse = jnp.log(jnp.sum(jnp.exp(logits - m), axis=-1, keepdims=True)) + m
    logp = logits - lse
    onehot = (lax.broadcasted_iota(jnp.int32, (RBLK, C), 1)
              == lab_ref[...]).astype(jnp.float32)
    picked = jnp.sum(logp * onehot, axis=-1, keepdims=True)
    rows = (i * RBLK
            + lax.broadcasted_iota(jnp.int32, (RBLK, 1), 0))
    is_src = rows < N
    cls_part = jnp.sum(jnp.where(is_src, picked, 0.0))

    h1 = h1_ref[...]
    dom = (jnp.dot(h1, dw_ref[0:H], preferred_element_type=jnp.float32)
           + jnp.dot(h2, dw_ref[H:2 * H], preferred_element_type=jnp.float32)
           + jnp.dot(logits, dw_ref[2 * H:2 * H + C],
                     preferred_element_type=jnp.float32)
           + db_ref[...])
    md = jnp.max(dom, axis=-1, keepdims=True)
    lsed = jnp.log(jnp.sum(jnp.exp(dom - md), axis=-1, keepdims=True)) + md
    dlogp = dom - lsed
    picked_d = jnp.where(is_src, dlogp[:, 0:1], dlogp[:, 1:2])
    dom_part = jnp.sum(picked_d)

    @pl.when(i == 0)
    def _init():
        cls_ref[...] = jnp.zeros((1, 1), jnp.float32)
        dom_ref[...] = jnp.zeros((1, 1), jnp.float32)

    cls_ref[...] += jnp.reshape(cls_part, (1, 1))
    dom_ref[...] += jnp.reshape(dom_part, (1, 1))

    @pl.when(i == nsteps - 1)
    def _fin():
        class_loss = -cls_ref[...] / N
        domain_loss = -dom_ref[...] / (2 * N)
        out_ref[...] = class_loss + domain_loss * 0.01


def _final(parts, nd_col, b2, fcW, fcb, dW, db, h1, labels2):
    grid = (2 * N) // RBLK
    outs = pl.pallas_call(
        _final_kernel,
        grid=(grid,),
        in_specs=[
            pl.BlockSpec((NC, RBLK, H), lambda i: (0, i, 0)),
            pl.BlockSpec((RBLK, 1), lambda i: (i, 0)),
            pl.BlockSpec((1, H), lambda i: (0, 0)),
            pl.BlockSpec((H, C), lambda i: (0, 0)),
            pl.BlockSpec((1, C), lambda i: (0, 0)),
            pl.BlockSpec((2 * H + C, 2), lambda i: (0, 0)),
            pl.BlockSpec((1, 2), lambda i: (0, 0)),
            pl.BlockSpec((RBLK, H), lambda i: (i, 0)),
            pl.BlockSpec((RBLK, 1), lambda i: (i, 0)),
        ],
        out_specs=(pl.BlockSpec((1, 1), lambda i: (0, 0)),
                   pl.BlockSpec((1, 1), lambda i: (0, 0)),
                   pl.BlockSpec((1, 1), lambda i: (0, 0))),
        out_shape=(jax.ShapeDtypeStruct((1, 1), jnp.float32),
                   jax.ShapeDtypeStruct((1, 1), jnp.float32),
                   jax.ShapeDtypeStruct((1, 1), jnp.float32)),
    )(parts, nd_col, b2, fcW, fcb, dW, db, h1, labels2)
    return outs[2][0, 0]


def kernel(features_s, labels_s, features_t, edge_index_s, edge_index_t,
           W1, b1, W2, b2, fcW, fcb, dW, db):
    src_s = edge_index_s[0].astype(jnp.int32)
    dst_s = edge_index_s[1].astype(jnp.int32)
    src_t = edge_index_t[0].astype(jnp.int32)
    dst_t = edge_index_t[1].astype(jnp.int32)

    zrows = jnp.zeros((WBR, D), jnp.float32)
    onesr = jnp.ones((CHUNK, D), jnp.float32)

    deg_parts = _bincount_call(src_s, dst_s, src_t, dst_t, onesr, zrows)
    norm4 = _norms(deg_parts)
    ns_col = jnp.concatenate([norm4[0], norm4[2]], axis=0)
    nd_col = jnp.concatenate([norm4[1], norm4[3]], axis=0)

    x2 = jnp.concatenate([features_s, features_t], axis=0)
    hp1 = _hp(x2, ns_col, W1)
    parts1 = _agg_call(hp1[:N], hp1[N:], src_s, dst_s, src_t, dst_t,
                       zrows)
    h1, hp2 = _mid(parts1, nd_col, ns_col, b1.reshape(1, H), W2)
    parts2 = _agg_call(hp2[:N], hp2[N:], src_s, dst_s, src_t, dst_t,
                       zrows)

    labels2 = jnp.concatenate(
        [labels_s.astype(jnp.int32), jnp.zeros((N,), jnp.int32)]
    ).reshape(2 * N, 1)
    return _final(parts2, nd_col, b2.reshape(1, H), fcW, fcb.reshape(1, C),
                  dW, db.reshape(1, 2), h1, labels2)


# speedup vs baseline: 8.5629x; 1.0104x over previous
import functools
import jax
import jax.numpy as jnp
from jax import lax
from jax.experimental import pallas as pl
from jax.experimental.pallas import tpu as pltpu
from jax.experimental.pallas import tpu_sc as plsc

N = 10000
E = 320000
D = 128
H = 128
C = 16
NP = 10240
NC = 2
NS = 16
EPT = E // (NC * NS)
CHUNK = 128
NFULL = EPT // CHUNK
TAIL = EPT - NFULL * CHUNK
WBR = 1000

_mesh = plsc.VectorSubcoreMesh(
    core_axis_name="c", subcore_axis_name="s", num_cores=NC, num_subcores=NS)


def _wid(cid, sid):
    return sid * NC + cid


def _bincount_body(src_s, dst_s, src_t, dst_t, ones_hbm, zrows_hbm, out,
                   deg_sh, idx_v, idx_tail_v, ones_v, sem):
    cid = lax.axis_index("c")
    sid = lax.axis_index("s")
    wid = _wid(cid, sid)

    pltpu.sync_copy(ones_hbm, ones_v)
    for a, arr in enumerate((src_s, dst_s, src_t, dst_t)):
        @pl.when(sid < 10)
        def _z(a=a):
            pltpu.sync_copy(zrows_hbm, deg_sh.at[pl.ds(sid * WBR, WBR)])
        plsc.subcore_barrier()

        base = wid * EPT

        def chunk_body(c, _, arr=arr, base=base):
            off = base + c * CHUNK
            pltpu.sync_copy(arr.at[pl.ds(off, CHUNK)], idx_v)
            pltpu.sync_copy(ones_v, deg_sh.at[idx_v], add=True)
            return _

        lax.fori_loop(0, NFULL, chunk_body, 0, unroll=False)
        off = base + NFULL * CHUNK
        pltpu.sync_copy(arr.at[pl.ds(off, TAIL)], idx_tail_v)
        pltpu.sync_copy(ones_v.at[pl.ds(0, TAIL)],
                        deg_sh.at[idx_tail_v], add=True)

        plsc.subcore_barrier()

        @pl.when(sid < 10)
        def _wb(a=a):
            r0 = sid * WBR
            pltpu.sync_copy(deg_sh.at[pl.ds(r0, WBR)],
                            out.at[cid, a, pl.ds(r0, WBR)])
        plsc.subcore_barrier()


_bincount_call = pl.kernel(
    _bincount_body,
    out_type=jax.ShapeDtypeStruct((NC, 4, N, D), jnp.float32),
    mesh=_mesh,
    scratch_types=[
        pltpu.VMEM_SHARED((N, D), jnp.float32),
        pltpu.VMEM((CHUNK,), jnp.int32),
        pltpu.VMEM((TAIL,), jnp.int32),
        pltpu.VMEM((CHUNK, D), jnp.float32),
        pltpu.SemaphoreType.DMA,
    ],
)


KC = 80
NCH = EPT // KC
UNROLL = 4


def _agg_body(hp_s, hp_t, src_s, dst_s, src_t, dst_t, zrows_hbm, out,
              agg_sh, srcb, dstb, rowsb, semi0, semi1,
              semg0, semg1, semg2, semg3, sems):
    cid = lax.axis_index("c")
    sid = lax.axis_index("s")
    wid = _wid(cid, sid)
    semi = (semi0, semi1)
    semg = (semg0, semg1, semg2, semg3)
    NI = NCH // UNROLL

    for g, (hp, sarr, darr) in ((0, (hp_s, src_s, dst_s)),
                                (1, (hp_t, src_t, dst_t))):
        @pl.when(sid < 10)
        def _z(g=g):
            pltpu.sync_copy(zrows_hbm,
                            agg_sh.at[pl.ds(sid * WBR, WBR)])
        plsc.subcore_barrier()

        base = wid * EPT

        def fire_idx(c, b, p, sarr=sarr, darr=darr, base=base):
            off = base + c * KC
            pltpu.async_copy(sarr.at[pl.ds(off, KC)],
                             srcb.at[p, b], semi[p])
            pltpu.async_copy(darr.at[pl.ds(off, KC)],
                             dstb.at[p, b], semi[p])

        def drain_idx(b, p, sarr=sarr):
            pltpu.make_async_copy(sarr.at[pl.ds(0, KC)],
                                  srcb.at[p, b], semi[p]).wait()
            pltpu.make_async_copy(sarr.at[pl.ds(0, KC)],
                                  dstb.at[p, b], semi[p]).wait()

        def drain_scatter():
            pltpu.make_async_copy(zrows_hbm.at[pl.ds(0, KC)],
                                  rowsb.at[0], sems).wait()

        def body(i, p, pn, last, hp=hp):
            c0 = i * UNROLL
            if not last:
                for u in range(UNROLL):
                    fire_idx(c0 + UNROLL + u, u, pn)
            else:
                fire_idx(c0 + UNROLL, 0, pn)
            for u in range(UNROLL):
                drain_idx(u, p)
            @pl.when(i > 0)
            def _ds():
                for u in range(UNROLL):
                    drain_scatter()
            gds = [pltpu.async_copy(hp.at[srcb.at[p, u]], rowsb.at[u],
                                    semg[u]) for u in range(UNROLL)]
            for u in range(UNROLL):
                gds[u].wait()
                pltpu.async_copy(rowsb.at[u], agg_sh.at[dstb.at[p, u]],
                                 sems, add=True)

        for u in range(UNROLL):
            fire_idx(u, u, 0)

        def pair_body(j, _):
            body(2 * j, 0, 1, False)
            body(2 * j + 1, 1, 0, False)
            return _

        lax.fori_loop(0, (NI - 1) // 2, pair_body, 0, unroll=False)
        body(NI - 1, 0, 1, True)
        drain_idx(0, 1)
        drain_scatter()
        pltpu.async_copy(hp.at[srcb.at[1, 0]], rowsb.at[0], semg[0]).wait()
        pltpu.async_copy(rowsb.at[0], agg_sh.at[dstb.at[1, 0]], sems,
                         add=True)
        for _u in range(UNROLL):
            drain_scatter()

        plsc.subcore_barrier()

        @pl.when(sid < 10)
        def _wb(g=g):
            r0 = sid * WBR
            pltpu.sync_copy(agg_sh.at[pl.ds(r0, WBR)],
                            out.at[cid, pl.ds(g * N + r0, WBR)])
        plsc.subcore_barrier()


_agg_call = pl.kernel(
    _agg_body,
    out_type=jax.ShapeDtypeStruct((NC, 2 * N, D), jnp.float32),
    mesh=_mesh,
    scratch_types=[
        pltpu.VMEM_SHARED((N, D), jnp.float32),
        pltpu.VMEM((2, UNROLL, KC), jnp.int32),
        pltpu.VMEM((2, UNROLL, KC), jnp.int32),
        pltpu.VMEM((UNROLL, KC, D), jnp.float32),
        pltpu.SemaphoreType.DMA,
        pltpu.SemaphoreType.DMA,
        pltpu.SemaphoreType.DMA,
        pltpu.SemaphoreType.DMA,
        pltpu.SemaphoreType.DMA,
        pltpu.SemaphoreType.DMA,
        pltpu.SemaphoreType.DMA,
    ],
)


def _norm_kernel(deg_ref, out_ref):
    deg = deg_ref[0, 0] + deg_ref[1, 0]
    degv = jnp.max(deg, axis=-1, keepdims=True)
    out_ref[...] = lax.rsqrt(jnp.maximum(degv, 1.0)).reshape(1, N, 1)


def _norms(deg_parts):
    return pl.pallas_call(
        _norm_kernel,
        grid=(4,),
        in_specs=[pl.BlockSpec((NC, 1, N, D), lambda a: (0, a, 0, 0))],
        out_specs=pl.BlockSpec((1, N, 1), lambda a: (a, 0, 0)),
        out_shape=jax.ShapeDtypeStruct((4, N, 1), jnp.float32),
    )(deg_parts)


RBLK = 2000


def _hp_kernel(x_ref, ns_ref, w_ref, hp_ref):
    x = x_ref[...]
    ns = ns_ref[...]
    hp_ref[...] = jnp.dot(x * ns, w_ref[...],
                          preferred_element_type=jnp.float32)


def _hp(x2, ns_col, w):
    grid = (2 * N) // RBLK
    return pl.pallas_call(
        _hp_kernel,
        grid=(grid,),
        in_specs=[
            pl.BlockSpec((RBLK, D), lambda i: (i, 0)),
            pl.BlockSpec((RBLK, 1), lambda i: (i, 0)),
            pl.BlockSpec((D, H), lambda i: (0, 0)),
        ],
        out_specs=pl.BlockSpec((RBLK, H), lambda i: (i, 0)),
        out_shape=jax.ShapeDtypeStruct((2 * N, H), jnp.float32),
    )(x2, ns_col, w)


def _mid_kernel(parts_ref, nd_ref, ns_ref, b_ref, w_ref, h_ref, hp_ref):
    agg = parts_ref[0] + parts_ref[1]
    h = jnp.maximum(agg * nd_ref[...] + b_ref[...], 0.0)
    h_ref[...] = h
    hp_ref[...] = jnp.dot(h * ns_ref[...], w_ref[...],
                          preferred_element_type=jnp.float32)


def _mid(parts, nd_col, ns_col, b, w):
    grid = (2 * N) // RBLK
    return pl.pallas_call(
        _mid_kernel,
        grid=(grid,),
        in_specs=[
            pl.BlockSpec((NC, RBLK, H), lambda i: (0, i, 0)),
            pl.BlockSpec((RBLK, 1), lambda i: (i, 0)),
            pl.BlockSpec((RBLK, 1), lambda i: (i, 0)),
            pl.BlockSpec((1, H), lambda i: (0, 0)),
            pl.BlockSpec((H, H), lambda i: (0, 0)),
        ],
        out_specs=(pl.BlockSpec((RBLK, H), lambda i: (i, 0)),
                   pl.BlockSpec((RBLK, H), lambda i: (i, 0))),
        out_shape=(jax.ShapeDtypeStruct((2 * N, H), jnp.float32),
                   jax.ShapeDtypeStruct((2 * N, H), jnp.float32)),
    )(parts, nd_col, ns_col, b, w)


def _final_kernel(parts_ref, nd_ref, b2_ref, fcw_ref, fcb_ref, dw_ref,
                  db_ref, h1_ref, lab_ref, cls_ref, dom_ref, out_ref):
    i = pl.program_id(0)
    nsteps = pl.num_programs(0)

    agg = parts_ref[0] + parts_ref[1]
    h2 = jnp.maximum(agg * nd_ref[...] + b2_ref[...], 0.0)
    logits = jnp.dot(h2, fcw_ref[...],
                     preferred_element_type=jnp.float32) + fcb_ref[...]
    m = jnp.max(logits, axis=-1, keepdims=True)
    lse = jnp.log(jnp.sum(jnp.exp(logits - m), axis=-1, keepdims=True)) + m
    logp = logits - lse
    onehot = (lax.broadcasted_iota(jnp.int32, (RBLK, C), 1)
              == lab_ref[...]).astype(jnp.float32)
    picked = jnp.sum(logp * onehot, axis=-1, keepdims=True)
    rows = (i * RBLK
            + lax.broadcasted_iota(jnp.int32, (RBLK, 1), 0))
    is_src = rows < N
    cls_part = jnp.sum(jnp.where(is_src, picked, 0.0))

    h1 = h1_ref[...]
    dom = (jnp.dot(h1, dw_ref[0:H], preferred_element_type=jnp.float32)
           + jnp.dot(h2, dw_ref[H:2 * H], preferred_element_type=jnp.float32)
           + jnp.dot(logits, dw_ref[2 * H:2 * H + C],
                     preferred_element_type=jnp.float32)
           + db_ref[...])
    md = jnp.max(dom, axis=-1, keepdims=True)
    lsed = jnp.log(jnp.sum(jnp.exp(dom - md), axis=-1, keepdims=True)) + md
    dlogp = dom - lsed
    picked_d = jnp.where(is_src, dlogp[:, 0:1], dlogp[:, 1:2])
    dom_part = jnp.sum(picked_d)

    @pl.when(i == 0)
    def _init():
        cls_ref[...] = jnp.zeros((1, 1), jnp.float32)
        dom_ref[...] = jnp.zeros((1, 1), jnp.float32)

    cls_ref[...] += jnp.reshape(cls_part, (1, 1))
    dom_ref[...] += jnp.reshape(dom_part, (1, 1))

    @pl.when(i == nsteps - 1)
    def _fin():
        class_loss = -cls_ref[...] / N
        domain_loss = -dom_ref[...] / (2 * N)
        out_ref[...] = class_loss + domain_loss * 0.01


def _final(parts, nd_col, b2, fcW, fcb, dW, db, h1, labels2):
    grid = (2 * N) // RBLK
    outs = pl.pallas_call(
        _final_kernel,
        grid=(grid,),
        in_specs=[
            pl.BlockSpec((NC, RBLK, H), lambda i: (0, i, 0)),
            pl.BlockSpec((RBLK, 1), lambda i: (i, 0)),
            pl.BlockSpec((1, H), lambda i: (0, 0)),
            pl.BlockSpec((H, C), lambda i: (0, 0)),
            pl.BlockSpec((1, C), lambda i: (0, 0)),
            pl.BlockSpec((2 * H + C, 2), lambda i: (0, 0)),
            pl.BlockSpec((1, 2), lambda i: (0, 0)),
            pl.BlockSpec((RBLK, H), lambda i: (i, 0)),
            pl.BlockSpec((RBLK, 1), lambda i: (i, 0)),
        ],
        out_specs=(pl.BlockSpec((1, 1), lambda i: (0, 0)),
                   pl.BlockSpec((1, 1), lambda i: (0, 0)),
                   pl.BlockSpec((1, 1), lambda i: (0, 0))),
        out_shape=(jax.ShapeDtypeStruct((1, 1), jnp.float32),
                   jax.ShapeDtypeStruct((1, 1), jnp.float32),
                   jax.ShapeDtypeStruct((1, 1), jnp.float32)),
    )(parts, nd_col, b2, fcW, fcb, dW, db, h1, labels2)
    return outs[2][0, 0]


def kernel(features_s, labels_s, features_t, edge_index_s, edge_index_t,
           W1, b1, W2, b2, fcW, fcb, dW, db):
    src_s = edge_index_s[0].astype(jnp.int32)
    dst_s = edge_index_s[1].astype(jnp.int32)
    src_t = edge_index_t[0].astype(jnp.int32)
    dst_t = edge_index_t[1].astype(jnp.int32)

    zrows = jnp.zeros((WBR, D), jnp.float32)
    onesr = jnp.ones((CHUNK, D), jnp.float32)

    deg_parts = _bincount_call(src_s, dst_s, src_t, dst_t, onesr, zrows)
    norm4 = _norms(deg_parts)
    ns_col = jnp.concatenate([norm4[0], norm4[2]], axis=0)
    nd_col = jnp.concatenate([norm4[1], norm4[3]], axis=0)

    x2 = jnp.concatenate([features_s, features_t], axis=0)
    hp1 = _hp(x2, ns_col, W1)
    parts1 = _agg_call(hp1[:N], hp1[N:], src_s, dst_s, src_t, dst_t,
                       zrows)
    h1, hp2 = _mid(parts1, nd_col, ns_col, b1.reshape(1, H), W2)
    parts2 = _agg_call(hp2[:N], hp2[N:], src_s, dst_s, src_t, dst_t,
                       zrows)

    labels2 = jnp.concatenate(
        [labels_s.astype(jnp.int32), jnp.zeros((N,), jnp.int32)]
    ).reshape(2 * N, 1)
    return _final(parts2, nd_col, b2.reshape(1, H), fcW, fcb.reshape(1, C),
                  dW, db.reshape(1, 2), h1, labels2)


# speedup vs baseline: 9.8096x; 1.1456x over previous
import functools
import jax
import jax.numpy as jnp
from jax import lax
from jax.experimental import pallas as pl
from jax.experimental.pallas import tpu as pltpu
from jax.experimental.pallas import tpu_sc as plsc

N = 10000
E = 320000
D = 128
H = 128
C = 16
NP = 10240
NC = 2
NS = 16
EPT = E // (NC * NS)
CHUNK = 128
NFULL = EPT // CHUNK
TAIL = EPT - NFULL * CHUNK
WBR = 1000
WBR16 = 2000

_mesh = plsc.VectorSubcoreMesh(
    core_axis_name="c", subcore_axis_name="s", num_cores=NC, num_subcores=NS)


def _wid(cid, sid):
    return sid * NC + cid


def _bincount_body(src_s, dst_s, src_t, dst_t, ones_hbm, zrows_hbm, out,
                   deg_sh, idx0, idx1, idx_tail_v, ones_v, sems):
    cid = lax.axis_index("c")
    sid = lax.axis_index("s")
    wid = _wid(cid, sid)
    idxb = (idx0, idx1)

    pltpu.sync_copy(ones_hbm, ones_v)
    for a, arr in enumerate((src_s, dst_s, src_t, dst_t)):
        @pl.when(sid < 10)
        def _z(a=a):
            pltpu.sync_copy(zrows_hbm, deg_sh.at[pl.ds(sid * WBR, WBR)])
        plsc.subcore_barrier()

        base = wid * EPT

        def drain_one():
            pltpu.make_async_copy(zrows_hbm.at[pl.ds(0, CHUNK)],
                                  ones_v, sems).wait()

        def chunk(c, p, arr=arr, base=base):
            @pl.when(c >= 2)
            def _d():
                drain_one()
            off = base + c * CHUNK
            pltpu.sync_copy(arr.at[pl.ds(off, CHUNK)], idxb[p])
            pltpu.async_copy(ones_v, deg_sh.at[idxb[p]], sems, add=True)

        def pair_body(j, _):
            chunk(2 * j, 0)
            chunk(2 * j + 1, 1)
            return _

        lax.fori_loop(0, NFULL // 2, pair_body, 0, unroll=False)
        off = base + NFULL * CHUNK
        pltpu.sync_copy(arr.at[pl.ds(off, TAIL)], idx_tail_v)
        pltpu.async_copy(ones_v.at[pl.ds(0, TAIL)],
                         deg_sh.at[idx_tail_v], sems, add=True)
        drain_one()
        drain_one()
        pltpu.make_async_copy(zrows_hbm.at[pl.ds(0, TAIL)],
                              ones_v.at[pl.ds(0, TAIL)],
                              sems).wait()
        plsc.subcore_barrier()

        @pl.when(sid < 10)
        def _wb(a=a):
            r0 = sid * WBR
            pltpu.sync_copy(deg_sh.at[pl.ds(r0, WBR)],
                            out.at[cid, a, pl.ds(r0, WBR)])
        plsc.subcore_barrier()


_bincount_call = pl.kernel(
    _bincount_body,
    out_type=jax.ShapeDtypeStruct((NC, 4, N, D), jnp.float32),
    mesh=_mesh,
    scratch_types=[
        pltpu.VMEM_SHARED((N, D), jnp.float32),
        pltpu.VMEM((CHUNK,), jnp.int32),
        pltpu.VMEM((CHUNK,), jnp.int32),
        pltpu.VMEM((TAIL,), jnp.int32),
        pltpu.VMEM((CHUNK, D), jnp.float32),
        pltpu.SemaphoreType.DMA,
    ],
)


KC = 80
NCH = EPT // KC
UNROLL = 4


def _agg_body(hp, src_s, dst_s, src_t, dst_t, zrows_hbm, out,
              agg_sh, srcb, dstb, rowsb, semi0, semi1,
              semg0, semg1, semg2, semg3, sems):
    cid = lax.axis_index("c")
    sid = lax.axis_index("s")
    wid = _wid(cid, sid)
    semi = (semi0, semi1)
    semg = (semg0, semg1, semg2, semg3)
    NI = NCH // UNROLL

    for g, (sarr, darr) in ((0, (src_s, dst_s)), (1, (src_t, dst_t))):
        @pl.when(sid < 10)
        def _z(g=g):
            pltpu.sync_copy(zrows_hbm,
                            agg_sh.at[pl.ds(sid * WBR, WBR)])
        plsc.subcore_barrier()

        base = wid * EPT

        def fire_idx(c, b, p, sarr=sarr, darr=darr, base=base):
            off = base + c * KC
            pltpu.async_copy(sarr.at[pl.ds(off, KC)],
                             srcb.at[p, b], semi[p])
            pltpu.async_copy(darr.at[pl.ds(off, KC)],
                             dstb.at[p, b], semi[p])

        def drain_idx(b, p, sarr=sarr):
            pltpu.make_async_copy(sarr.at[pl.ds(0, KC)],
                                  srcb.at[p, b], semi[p]).wait()
            pltpu.make_async_copy(sarr.at[pl.ds(0, KC)],
                                  dstb.at[p, b], semi[p]).wait()

        def drain_scatter():
            pltpu.make_async_copy(zrows_hbm.at[pl.ds(0, KC)],
                                  rowsb.at[0], sems).wait()

        def body(i, p, pn, last):
            c0 = i * UNROLL
            if not last:
                for u in range(UNROLL):
                    fire_idx(c0 + UNROLL + u, u, pn)
            else:
                fire_idx(c0 + UNROLL, 0, pn)
            for u in range(UNROLL):
                drain_idx(u, p)
            @pl.when(i > 0)
            def _ds():
                for u in range(UNROLL):
                    drain_scatter()
            gds = [pltpu.async_copy(hp.at[srcb.at[p, u]], rowsb.at[u],
                                    semg[u]) for u in range(UNROLL)]
            for u in range(UNROLL):
                gds[u].wait()
                pltpu.async_copy(rowsb.at[u], agg_sh.at[dstb.at[p, u]],
                                 sems, add=True)

        for u in range(UNROLL):
            fire_idx(u, u, 0)

        def pair_body(j, _):
            body(2 * j, 0, 1, False)
            body(2 * j + 1, 1, 0, False)
            return _

        lax.fori_loop(0, (NI - 1) // 2, pair_body, 0, unroll=False)
        body(NI - 1, 0, 1, True)
        drain_idx(0, 1)
        drain_scatter()
        pltpu.async_copy(hp.at[srcb.at[1, 0]], rowsb.at[0], semg[0]).wait()
        pltpu.async_copy(rowsb.at[0], agg_sh.at[dstb.at[1, 0]], sems,
                         add=True)
        for _u in range(UNROLL):
            drain_scatter()

        plsc.subcore_barrier()

        @pl.when(sid < 10)
        def _wb(g=g):
            r0 = sid * WBR
            pltpu.sync_copy(agg_sh.at[pl.ds(r0, WBR)],
                            out.at[cid, pl.ds(g * N + r0, WBR)])
        plsc.subcore_barrier()


_agg_call = pl.kernel(
    _agg_body,
    out_type=jax.ShapeDtypeStruct((NC, 2 * N, D), jnp.float32),
    mesh=_mesh,
    scratch_types=[
        pltpu.VMEM_SHARED((N, D), jnp.float32),
        pltpu.VMEM((2, UNROLL, KC), jnp.int32),
        pltpu.VMEM((2, UNROLL, KC), jnp.int32),
        pltpu.VMEM((UNROLL, KC, D), jnp.float32),
        pltpu.SemaphoreType.DMA,
        pltpu.SemaphoreType.DMA,
        pltpu.SemaphoreType.DMA,
        pltpu.SemaphoreType.DMA,
        pltpu.SemaphoreType.DMA,
        pltpu.SemaphoreType.DMA,
        pltpu.SemaphoreType.DMA,
    ],
)


def _norm_kernel(deg_ref, out_ref):
    deg = deg_ref[0, 0] + deg_ref[1, 0]
    degv = jnp.max(deg, axis=-1, keepdims=True)
    out_ref[...] = lax.rsqrt(jnp.maximum(degv, 1.0)).reshape(1, N, 1)


def _norms(deg_parts):
    return pl.pallas_call(
        _norm_kernel,
        grid=(4,),
        in_specs=[pl.BlockSpec((NC, 1, N, D), lambda a: (0, a, 0, 0))],
        out_specs=pl.BlockSpec((1, N, 1), lambda a: (a, 0, 0)),
        out_shape=jax.ShapeDtypeStruct((4, N, 1), jnp.float32),
    )(deg_parts)


RBLK = 2000


def _hp_kernel(x_ref, ns_ref, w_ref, hp_ref):
    x = x_ref[...]
    ns = ns_ref[...]
    hp_ref[...] = jnp.dot(x * ns, w_ref[...],
                          preferred_element_type=jnp.float32)


def _hp(x2, ns_col, w):
    grid = (2 * N) // RBLK
    return pl.pallas_call(
        _hp_kernel,
        grid=(grid,),
        in_specs=[
            pl.BlockSpec((RBLK, D), lambda i: (i, 0)),
            pl.BlockSpec((RBLK, 1), lambda i: (i, 0)),
            pl.BlockSpec((D, H), lambda i: (0, 0)),
        ],
        out_specs=pl.BlockSpec((RBLK, H), lambda i: (i, 0)),
        out_shape=jax.ShapeDtypeStruct((2 * N, H), jnp.float32),
    )(x2, ns_col, w)


def _mid_kernel(parts_ref, nd_ref, ns_ref, b_ref, w_ref, h_ref, hp_ref):
    agg = parts_ref[0] + parts_ref[1]
    h = jnp.maximum(agg * nd_ref[...] + b_ref[...], 0.0)
    h_ref[...] = h
    hp_ref[...] = jnp.dot(h * ns_ref[...], w_ref[...],
                          preferred_element_type=jnp.float32)


def _mid(parts, nd_col, ns_col, b, w):
    grid = (2 * N) // RBLK
    return pl.pallas_call(
        _mid_kernel,
        grid=(grid,),
        in_specs=[
            pl.BlockSpec((NC, RBLK, H), lambda i: (0, i, 0)),
            pl.BlockSpec((RBLK, 1), lambda i: (i, 0)),
            pl.BlockSpec((RBLK, 1), lambda i: (i, 0)),
            pl.BlockSpec((1, H), lambda i: (0, 0)),
            pl.BlockSpec((H, H), lambda i: (0, 0)),
        ],
        out_specs=(pl.BlockSpec((RBLK, H), lambda i: (i, 0)),
                   pl.BlockSpec((RBLK, H), lambda i: (i, 0))),
        out_shape=(jax.ShapeDtypeStruct((2 * N, H), jnp.float32),
                   jax.ShapeDtypeStruct((2 * N, H), jnp.float32)),
    )(parts, nd_col, ns_col, b, w)


def _final_kernel(parts_ref, nd_ref, b2_ref, fcw_ref, fcb_ref, dw_ref,
                  db_ref, h1_ref, lab_ref, cls_ref, dom_ref, out_ref):
    i = pl.program_id(0)
    nsteps = pl.num_programs(0)

    agg = parts_ref[0] + parts_ref[1]
    h2 = jnp.maximum(agg * nd_ref[...] + b2_ref[...], 0.0)
    logits = jnp.dot(h2, fcw_ref[...],
                     preferred_element_type=jnp.float32) + fcb_ref[...]
    m = jnp.max(logits, axis=-1, keepdims=True)
    lse = jnp.log(jnp.sum(jnp.exp(logits - m), axis=-1, keepdims=True)) + m
    logp = logits - lse
    onehot = (lax.broadcasted_iota(jnp.int32, (RBLK, C), 1)
              == lab_ref[...]).astype(jnp.float32)
    picked = jnp.sum(logp * onehot, axis=-1, keepdims=True)
    rows = (i * RBLK
            + lax.broadcasted_iota(jnp.int32, (RBLK, 1), 0))
    is_src = rows < N
    cls_part = jnp.sum(jnp.where(is_src, picked, 0.0))

    h1 = h1_ref[...]
    dom = (jnp.dot(h1, dw_ref[0:H], preferred_element_type=jnp.float32)
           + jnp.dot(h2, dw_ref[H:2 * H], preferred_element_type=jnp.float32)
           + jnp.dot(logits, dw_ref[2 * H:2 * H + C],
                     preferred_element_type=jnp.float32)
           + db_ref[...])
    md = jnp.max(dom, axis=-1, keepdims=True)
    lsed = jnp.log(jnp.sum(jnp.exp(dom - md), axis=-1, keepdims=True)) + md
    dlogp = dom - lsed
    picked_d = jnp.where(is_src, dlogp[:, 0:1], dlogp[:, 1:2])
    dom_part = jnp.sum(picked_d)

    @pl.when(i == 0)
    def _init():
        cls_ref[...] = jnp.zeros((1, 1), jnp.float32)
        dom_ref[...] = jnp.zeros((1, 1), jnp.float32)

    cls_ref[...] += jnp.reshape(cls_part, (1, 1))
    dom_ref[...] += jnp.reshape(dom_part, (1, 1))

    @pl.when(i == nsteps - 1)
    def _fin():
        class_loss = -cls_ref[...] / N
        domain_loss = -dom_ref[...] / (2 * N)
        out_ref[...] = class_loss + domain_loss * 0.01


def _final(parts, nd_col, b2, fcW, fcb, dW, db, h1, labels2):
    grid = (2 * N) // RBLK
    outs = pl.pallas_call(
        _final_kernel,
        grid=(grid,),
        in_specs=[
            pl.BlockSpec((NC, RBLK, H), lambda i: (0, i, 0)),
            pl.BlockSpec((RBLK, 1), lambda i: (i, 0)),
            pl.BlockSpec((1, H), lambda i: (0, 0)),
            pl.BlockSpec((H, C), lambda i: (0, 0)),
            pl.BlockSpec((1, C), lambda i: (0, 0)),
            pl.BlockSpec((2 * H + C, 2), lambda i: (0, 0)),
            pl.BlockSpec((1, 2), lambda i: (0, 0)),
            pl.BlockSpec((RBLK, H), lambda i: (i, 0)),
            pl.BlockSpec((RBLK, 1), lambda i: (i, 0)),
        ],
        out_specs=(pl.BlockSpec((1, 1), lambda i: (0, 0)),
                   pl.BlockSpec((1, 1), lambda i: (0, 0)),
                   pl.BlockSpec((1, 1), lambda i: (0, 0))),
        out_shape=(jax.ShapeDtypeStruct((1, 1), jnp.float32),
                   jax.ShapeDtypeStruct((1, 1), jnp.float32),
                   jax.ShapeDtypeStruct((1, 1), jnp.float32)),
    )(parts, nd_col, b2, fcW, fcb, dW, db, h1, labels2)
    return outs[2][0, 0]


def kernel(features_s, labels_s, features_t, edge_index_s, edge_index_t,
           W1, b1, W2, b2, fcW, fcb, dW, db):
    src_s = edge_index_s[0].astype(jnp.int32)
    dst_s = edge_index_s[1].astype(jnp.int32)
    src_t = edge_index_t[0].astype(jnp.int32)
    dst_t = edge_index_t[1].astype(jnp.int32)

    zrows = jnp.zeros((WBR, D), jnp.float32)
    onesr = jnp.ones((CHUNK, D), jnp.float32)

    deg_parts = _bincount_call(src_s, dst_s, src_t, dst_t, onesr, zrows)
    norm4 = _norms(deg_parts)
    ns_col = jnp.concatenate([norm4[0], norm4[2]], axis=0)
    nd_col = jnp.concatenate([norm4[1], norm4[3]], axis=0)

    x2 = jnp.concatenate([features_s, features_t], axis=0)
    hp1 = _hp(x2, ns_col, W1)
    src_t_off = src_t + N
    parts1 = _agg_call(hp1, src_s, dst_s, src_t_off, dst_t, zrows)
    h1, hp2 = _mid(parts1, nd_col, ns_col, b1.reshape(1, H), W2)
    parts2 = _agg_call(hp2, src_s, dst_s, src_t_off, dst_t, zrows)

    labels2 = jnp.concatenate(
        [labels_s.astype(jnp.int32), jnp.zeros((N,), jnp.int32)]
    ).reshape(2 * N, 1)
    return _final(parts2, nd_col, b2.reshape(1, H), fcW, fcb.reshape(1, C),
                  dW, db.reshape(1, 2), h1, labels2)


# speedup vs baseline: 10.2403x; 1.0439x over previous
import jax
import jax.numpy as jnp
from jax import lax
from jax.experimental import pallas as pl
from jax.experimental.pallas import tpu as pltpu
from jax.experimental.pallas import tpu_sc as plsc

N = 10000
E = 320000
D = 128
H = 128
C = 16
NC = 2
NS = 16
EPT = E // (NC * NS)
CHUNK = 128
NFULL = EPT // CHUNK
TAIL = EPT - NFULL * CHUNK
WBR = 1000

_mesh = plsc.VectorSubcoreMesh(
    core_axis_name="c", subcore_axis_name="s", num_cores=NC, num_subcores=NS)


def _wid(cid, sid):
    return sid * NC + cid


def _bincount_body(src_s, dst_s, src_t, dst_t, ones_hbm, zrows_hbm, out,
                   deg_sh, idx0, idx1, idx_tail_v, ones_v, sems):
    cid = lax.axis_index("c")
    sid = lax.axis_index("s")
    wid = _wid(cid, sid)
    idxb = (idx0, idx1)

    pltpu.sync_copy(ones_hbm, ones_v)
    for a, arr in enumerate((src_s, dst_s, src_t, dst_t)):
        @pl.when(sid < 10)
        def _z(a=a):
            pltpu.sync_copy(zrows_hbm, deg_sh.at[pl.ds(sid * WBR, WBR)])
        plsc.subcore_barrier()

        base = wid * EPT

        def drain_one():
            pltpu.make_async_copy(zrows_hbm.at[pl.ds(0, CHUNK)],
                                  ones_v, sems).wait()

        def chunk(c, p, arr=arr, base=base):
            @pl.when(c >= 2)
            def _d():
                drain_one()
            off = base + c * CHUNK
            pltpu.sync_copy(arr.at[pl.ds(off, CHUNK)], idxb[p])
            pltpu.async_copy(ones_v, deg_sh.at[idxb[p]], sems, add=True)

        def pair_body(j, _):
            chunk(2 * j, 0)
            chunk(2 * j + 1, 1)
            return _

        lax.fori_loop(0, NFULL // 2, pair_body, 0, unroll=False)
        off = base + NFULL * CHUNK
        pltpu.sync_copy(arr.at[pl.ds(off, TAIL)], idx_tail_v)
        pltpu.async_copy(ones_v.at[pl.ds(0, TAIL)],
                         deg_sh.at[idx_tail_v], sems, add=True)
        drain_one()
        drain_one()
        pltpu.make_async_copy(zrows_hbm.at[pl.ds(0, TAIL)],
                              ones_v.at[pl.ds(0, TAIL)],
                              sems).wait()
        plsc.subcore_barrier()

        @pl.when(sid < 10)
        def _wb(a=a):
            r0 = sid * WBR
            pltpu.sync_copy(deg_sh.at[pl.ds(r0, WBR)],
                            out.at[cid, a, pl.ds(r0, WBR)])
        plsc.subcore_barrier()


_bincount_call = pl.kernel(
    _bincount_body,
    out_type=jax.ShapeDtypeStruct((NC, 4, N, D), jnp.float32),
    mesh=_mesh,
    scratch_types=[
        pltpu.VMEM_SHARED((N, D), jnp.float32),
        pltpu.VMEM((CHUNK,), jnp.int32),
        pltpu.VMEM((CHUNK,), jnp.int32),
        pltpu.VMEM((TAIL,), jnp.int32),
        pltpu.VMEM((CHUNK, D), jnp.float32),
        pltpu.SemaphoreType.DMA,
    ],
)


KC = 80
NCH = EPT // KC
UNROLL = 4


def _agg_body(hp, src_s, dst_s, src_t, dst_t, zrows_hbm, out,
              agg_sh, srcb, dstb, rowsb, semi0, semi1,
              semg0, semg1, semg2, semg3, sems):
    cid = lax.axis_index("c")
    sid = lax.axis_index("s")
    wid = _wid(cid, sid)
    semi = (semi0, semi1)
    semg = (semg0, semg1, semg2, semg3)
    NI = NCH // UNROLL

    for g, (sarr, darr) in ((0, (src_s, dst_s)), (1, (src_t, dst_t))):
        @pl.when(sid < 10)
        def _z(g=g):
            pltpu.sync_copy(zrows_hbm,
                            agg_sh.at[pl.ds(sid * WBR, WBR)])
        plsc.subcore_barrier()

        base = wid * EPT

        def fire_idx(c, b, p, sarr=sarr, darr=darr, base=base):
            off = base + c * KC
            pltpu.async_copy(sarr.at[pl.ds(off, KC)],
                             srcb.at[p, b], semi[p])
            pltpu.async_copy(darr.at[pl.ds(off, KC)],
                             dstb.at[p, b], semi[p])

        def drain_idx(b, p, sarr=sarr):
            pltpu.make_async_copy(sarr.at[pl.ds(0, KC)],
                                  srcb.at[p, b], semi[p]).wait()
            pltpu.make_async_copy(sarr.at[pl.ds(0, KC)],
                                  dstb.at[p, b], semi[p]).wait()

        def drain_scatter():
            pltpu.make_async_copy(zrows_hbm.at[pl.ds(0, KC)],
                                  rowsb.at[0], sems).wait()

        def body(i, p, pn, last):
            c0 = i * UNROLL
            if not last:
                for u in range(UNROLL):
                    fire_idx(c0 + UNROLL + u, u, pn)
            else:
                fire_idx(c0 + UNROLL, 0, pn)
            for u in range(UNROLL):
                drain_idx(u, p)
            @pl.when(i > 0)
            def _ds():
                for u in range(UNROLL):
                    drain_scatter()
            gds = [pltpu.async_copy(hp.at[srcb.at[p, u]], rowsb.at[u],
                                    semg[u]) for u in range(UNROLL)]
            for u in range(UNROLL):
                gds[u].wait()
                pltpu.async_copy(rowsb.at[u], agg_sh.at[dstb.at[p, u]],
                                 sems, add=True)

        for u in range(UNROLL):
            fire_idx(u, u, 0)

        def pair_body(j, _):
            body(2 * j, 0, 1, False)
            body(2 * j + 1, 1, 0, False)
            return _

        lax.fori_loop(0, (NI - 1) // 2, pair_body, 0, unroll=False)
        body(NI - 1, 0, 1, True)
        drain_idx(0, 1)
        drain_scatter()
        pltpu.async_copy(hp.at[srcb.at[1, 0]], rowsb.at[0], semg[0]).wait()
        pltpu.async_copy(rowsb.at[0], agg_sh.at[dstb.at[1, 0]], sems,
                         add=True)
        for _u in range(UNROLL):
            drain_scatter()

        plsc.subcore_barrier()

        @pl.when(sid < 10)
        def _wb(g=g):
            r0 = sid * WBR
            pltpu.sync_copy(agg_sh.at[pl.ds(r0, WBR)],
                            out.at[cid, pl.ds(g * N + r0, WBR)])
        plsc.subcore_barrier()


_agg_call = pl.kernel(
    _agg_body,
    out_type=jax.ShapeDtypeStruct((NC, 2 * N, D), jnp.float32),
    mesh=_mesh,
    scratch_types=[
        pltpu.VMEM_SHARED((N, D), jnp.float32),
        pltpu.VMEM((2, UNROLL, KC), jnp.int32),
        pltpu.VMEM((2, UNROLL, KC), jnp.int32),
        pltpu.VMEM((UNROLL, KC, D), jnp.float32),
        pltpu.SemaphoreType.DMA,
        pltpu.SemaphoreType.DMA,
        pltpu.SemaphoreType.DMA,
        pltpu.SemaphoreType.DMA,
        pltpu.SemaphoreType.DMA,
        pltpu.SemaphoreType.DMA,
        pltpu.SemaphoreType.DMA,
    ],
)


def _norm_of(deg_blk):
    deg = deg_blk[0, 0] + deg_blk[1, 0]
    degv = jnp.max(deg, axis=-1, keepdims=True)
    return lax.rsqrt(jnp.maximum(degv, 1.0))


def _deg_spec(which):
    return pl.BlockSpec((NC, 1, RBLK, D),
                        lambda i: (0, 2 * (i // 5) + which, i % 5, 0))


RBLK = 2000


def _hp_kernel(x_ref, degs_ref, w_ref, hp_ref):
    ns = _norm_of(degs_ref[...])
    hp_ref[...] = jnp.dot(x_ref[...] * ns, w_ref[...],
                          preferred_element_type=jnp.float32)


def _hp(x2, deg_parts, w):
    grid = (2 * N) // RBLK
    return pl.pallas_call(
        _hp_kernel,
        grid=(grid,),
        in_specs=[
            pl.BlockSpec((RBLK, D), lambda i: (i, 0)),
            _deg_spec(0),
            pl.BlockSpec((D, H), lambda i: (0, 0)),
        ],
        out_specs=pl.BlockSpec((RBLK, H), lambda i: (i, 0)),
        out_shape=jax.ShapeDtypeStruct((2 * N, H), jnp.float32),
    )(x2, deg_parts, w)


def _mid_kernel(parts_ref, degd_ref, degs_ref, b_ref, w_ref, h_ref,
                hp_ref):
    agg = parts_ref[0] + parts_ref[1]
    nd = _norm_of(degd_ref[...])
    ns = _norm_of(degs_ref[...])
    h = jnp.maximum(agg * nd + b_ref[...], 0.0)
    h_ref[...] = h
    hp_ref[...] = jnp.dot(h * ns, w_ref[...],
                          preferred_element_type=jnp.float32)


def _mid(parts, deg_parts, b, w):
    grid = (2 * N) // RBLK
    return pl.pallas_call(
        _mid_kernel,
        grid=(grid,),
        in_specs=[
            pl.BlockSpec((NC, RBLK, H), lambda i: (0, i, 0)),
            _deg_spec(1),
            _deg_spec(0),
            pl.BlockSpec((1, H), lambda i: (0, 0)),
            pl.BlockSpec((H, H), lambda i: (0, 0)),
        ],
        out_specs=(pl.BlockSpec((RBLK, H), lambda i: (i, 0)),
                   pl.BlockSpec((RBLK, H), lambda i: (i, 0))),
        out_shape=(jax.ShapeDtypeStruct((2 * N, H), jnp.float32),
                   jax.ShapeDtypeStruct((2 * N, H), jnp.float32)),
    )(parts, deg_parts, deg_parts, b, w)


def _final_kernel(parts_ref, degd_ref, b2_ref, fcw_ref, fcb_ref, dw_ref,
                  db_ref, h1_ref, lab_ref, cls_ref, dom_ref, out_ref):
    i = pl.program_id(0)
    nsteps = pl.num_programs(0)

    agg = parts_ref[0] + parts_ref[1]
    nd = _norm_of(degd_ref[...])
    h2 = jnp.maximum(agg * nd + b2_ref[...], 0.0)
    logits = jnp.dot(h2, fcw_ref[...],
                     preferred_element_type=jnp.float32) + fcb_ref[...]
    m = jnp.max(logits, axis=-1, keepdims=True)
    lse = jnp.log(jnp.sum(jnp.exp(logits - m), axis=-1, keepdims=True)) + m
    logp = logits - lse
    onehot = (lax.broadcasted_iota(jnp.int32, (RBLK, C), 1)
              == lab_ref[...]).astype(jnp.float32)
    picked = jnp.sum(logp * onehot, axis=-1, keepdims=True)
    rows = (i * RBLK
            + lax.broadcasted_iota(jnp.int32, (RBLK, 1), 0))
    is_src = rows < N
    cls_part = jnp.sum(jnp.where(is_src, picked, 0.0))

    h1 = h1_ref[...]
    dom = (jnp.dot(h1, dw_ref[0:H], preferred_element_type=jnp.float32)
           + jnp.dot(h2, dw_ref[H:2 * H], preferred_element_type=jnp.float32)
           + jnp.dot(logits, dw_ref[2 * H:2 * H + C],
                     preferred_element_type=jnp.float32)
           + db_ref[...])
    md = jnp.max(dom, axis=-1, keepdims=True)
    lsed = jnp.log(jnp.sum(jnp.exp(dom - md), axis=-1, keepdims=True)) + md
    dlogp = dom - lsed
    picked_d = jnp.where(is_src, dlogp[:, 0:1], dlogp[:, 1:2])
    dom_part = jnp.sum(picked_d)

    @pl.when(i == 0)
    def _init():
        cls_ref[...] = jnp.zeros((1, 1), jnp.float32)
        dom_ref[...] = jnp.zeros((1, 1), jnp.float32)

    cls_ref[...] += jnp.reshape(cls_part, (1, 1))
    dom_ref[...] += jnp.reshape(dom_part, (1, 1))

    @pl.when(i == nsteps - 1)
    def _fin():
        class_loss = -cls_ref[...] / N
        domain_loss = -dom_ref[...] / (2 * N)
        out_ref[...] = class_loss + domain_loss * 0.01


def _final(parts, deg_parts, b2, fcW, fcb, dW, db, h1, labels2):
    grid = (2 * N) // RBLK
    outs = pl.pallas_call(
        _final_kernel,
        grid=(grid,),
        in_specs=[
            pl.BlockSpec((NC, RBLK, H), lambda i: (0, i, 0)),
            _deg_spec(1),
            pl.BlockSpec((1, H), lambda i: (0, 0)),
            pl.BlockSpec((H, C), lambda i: (0, 0)),
            pl.BlockSpec((1, C), lambda i: (0, 0)),
            pl.BlockSpec((2 * H + C, 2), lambda i: (0, 0)),
            pl.BlockSpec((1, 2), lambda i: (0, 0)),
            pl.BlockSpec((RBLK, H), lambda i: (i, 0)),
            pl.BlockSpec((RBLK, 1), lambda i: (i, 0)),
        ],
        out_specs=(pl.BlockSpec((1, 1), lambda i: (0, 0)),
                   pl.BlockSpec((1, 1), lambda i: (0, 0)),
                   pl.BlockSpec((1, 1), lambda i: (0, 0))),
        out_shape=(jax.ShapeDtypeStruct((1, 1), jnp.float32),
                   jax.ShapeDtypeStruct((1, 1), jnp.float32),
                   jax.ShapeDtypeStruct((1, 1), jnp.float32)),
    )(parts, deg_parts, b2, fcW, fcb, dW, db, h1, labels2)
    return outs[2][0, 0]


def kernel(features_s, labels_s, features_t, edge_index_s, edge_index_t,
           W1, b1, W2, b2, fcW, fcb, dW, db):
    src_s = edge_index_s[0].astype(jnp.int32)
    dst_s = edge_index_s[1].astype(jnp.int32)
    src_t = edge_index_t[0].astype(jnp.int32)
    dst_t = edge_index_t[1].astype(jnp.int32)

    zrows = jnp.zeros((WBR, D), jnp.float32)
    onesr = jnp.ones((CHUNK, D), jnp.float32)

    deg_parts = _bincount_call(src_s, dst_s, src_t, dst_t, onesr, zrows)

    x2 = jnp.concatenate([features_s, features_t], axis=0)
    hp1 = _hp(x2, deg_parts, W1)
    src_t_off = src_t + N
    parts1 = _agg_call(hp1, src_s, dst_s, src_t_off, dst_t, zrows)
    h1, hp2 = _mid(parts1, deg_parts, b1.reshape(1, H), W2)
    parts2 = _agg_call(hp2, src_s, dst_s, src_t_off, dst_t, zrows)

    labels2 = jnp.concatenate(
        [labels_s.astype(jnp.int32), jnp.zeros((N,), jnp.int32)]
    ).reshape(2 * N, 1)
    return _final(parts2, deg_parts, b2.reshape(1, H), fcW,
                  fcb.reshape(1, C), dW, db.reshape(1, 2), h1, labels2)
